# baseline (speedup 1.0000x reference)
import functools

import jax
import jax.numpy as jnp
from jax import lax
from jax.experimental import pallas as pl
from jax.experimental.pallas import tpu as pltpu
from jax.experimental.pallas import tpu_sc as plsc

NC = 2
NS = 16
NW = NC * NS
LANES = 16
IDX_PER_STREAM = 128
HALF = 32


def _tree_sum(vals):
    while len(vals) > 1:
        nxt = []
        for i in range(0, len(vals) - 1, 2):
            nxt.append(vals[i] + vals[i + 1])
        if len(vals) % 2:
            nxt.append(vals[-1])
        vals = nxt
    return vals[0]


def _make_sc_kernel(B, L, V, D, CH):
    b_per_w = B // NW
    n_idx = b_per_w * L
    n_chunks = b_per_w // CH
    rows_per_chunk = CH * L
    steps_per_chunk = rows_per_chunk // IDX_PER_STREAM
    inv_l = 1.0 / L

    mesh = plsc.VectorSubcoreMesh(core_axis_name="c", subcore_axis_name="s")

    @functools.partial(
        pl.kernel,
        mesh=mesh,
        out_type=jax.ShapeDtypeStruct((B, D), jnp.float32),
        compiler_params=pltpu.CompilerParams(use_tc_tiling_on_sc=False),
        scratch_types=[
            pltpu.VMEM((n_idx,), jnp.int32),
            pltpu.VMEM((n_idx,), jnp.int32),
            pltpu.VMEM((n_idx,), jnp.int32),
            pltpu.VMEM((2, rows_per_chunk, HALF), jnp.float32),
            pltpu.VMEM((2, rows_per_chunk, HALF), jnp.float32),
            pltpu.VMEM((2, CH, D), jnp.float32),
            pltpu.SemaphoreType.DMA,
            pltpu.SemaphoreType.DMA,
            pltpu.SemaphoreType.DMA,
        ],
    )
    def sc_kernel(ctx_hbm, table_hbm, out_hbm, ctx_raw, ia_v, ib_v,
                  rows_a, rows_b, ob_v, sem0, sem1, out_sem):
        wid = lax.axis_index("s") * NC + lax.axis_index("c")
        pltpu.sync_copy(ctx_hbm.at[wid], ctx_raw)
        tab32 = table_hbm
        sems = (sem0, sem1)

        def prep_body(k, carry):
            sl = pl.ds(k * LANES, LANES)
            v = ctx_raw[sl]
            a = v + v
            ia_v[sl] = a
            ib_v[sl] = a + 1
            return carry

        lax.fori_loop(0, n_idx // LANES, prep_body, 0, unroll=False)

        def gather_copies(c, par):
            for s in range(steps_per_chunk):
                off = c * rows_per_chunk + s * IDX_PER_STREAM
                dst = pl.ds(s * IDX_PER_STREAM, IDX_PER_STREAM)
                yield pltpu.make_async_copy(
                    tab32.at[ia_v.at[pl.ds(off, IDX_PER_STREAM)]],
                    rows_a.at[par].at[dst], sems[par])
                yield pltpu.make_async_copy(
                    tab32.at[ib_v.at[pl.ds(off, IDX_PER_STREAM)]],
                    rows_b.at[par].at[dst], sems[par])

        def fire(c, par):
            for cp in gather_copies(c, par):
                cp.start()

        def drain(c, par):
            for cp in gather_copies(c, par):
                cp.wait()

        def accumulate(c, par):
            @pl.when(c >= 2)
            def _():
                pltpu.make_async_copy(
                    ob_v.at[par], out_hbm.at[pl.ds(0, CH)], out_sem
                ).wait()

            def e_body(e, carry2):
                base = e * L
                for half, rows in ((0, rows_a), (1, rows_b)):
                    for cg in range(HALF // LANES):
                        sl = pl.ds(cg * LANES, LANES)
                        vals = [rows[par, base + j, sl] for j in range(L)]
                        osl = pl.ds((half * (HALF // LANES) + cg) * LANES,
                                    LANES)
                        ob_v[par, e, osl] = _tree_sum(vals) * inv_l
                return carry2

            lax.fori_loop(0, CH, e_body, 0, unroll=False)
            out_base = wid * b_per_w + c * CH
            pltpu.make_async_copy(
                ob_v.at[par], out_hbm.at[pl.ds(out_base, CH)], out_sem
            ).start()

        fire(0, 0)

        def pair_body(p, carry):
            c = p * 2
            fire(c + 1, 1)
            drain(c, 0)
            accumulate(c, 0)

            @pl.when(c + 2 < n_chunks)
            def _():
                fire(c + 2, 0)

            drain(c + 1, 1)
            accumulate(c + 1, 1)
            return carry

        lax.fori_loop(0, n_chunks // 2, pair_body, 0, unroll=False)
        for par in range(2):
            pltpu.make_async_copy(
                ob_v.at[par], out_hbm.at[pl.ds(0, CH)], out_sem
            ).wait()

    return sc_kernel


@jax.jit
def kernel(contexts, table):
    B, L = contexts.shape
    V, D = table.shape
    CH = 32
    assert (CH * L) % IDX_PER_STREAM == 0
    assert B % (NW * CH) == 0
    assert D == 2 * HALF
    ctx2 = contexts.reshape(NW, (B // NW) * L)
    tab32 = table.reshape(V * 2, D // 2)
    return _make_sc_kernel(B, L, V, D, CH)(ctx2, tab32)
